# baseline (speedup 1.0000x reference)
import functools

import jax
import jax.numpy as jnp
from jax import lax
from jax.experimental import pallas as pl
from jax.experimental.pallas import tpu as pltpu
from jax.experimental.pallas import tpu_sc as plsc

NUM_TABLES = 3
EMB_DIM = 64
BATCH_SIZE = 16384
NUM_WORKERS = 32
ROWS_PER_WORKER = BATCH_SIZE // NUM_WORKERS
CHUNK = 128
CHUNKS_PER_WORKER = ROWS_PER_WORKER // CHUNK
LANES = 16


def _gather_first(idx_hbm, t_hbm, out_hbm, idx_v, rows, sem):
    wid = lax.axis_index("s") * 2 + lax.axis_index("c")
    pltpu.sync_copy(idx_hbm.at[wid], idx_v)
    for c in range(CHUNKS_PER_WORKER):
        pltpu.async_copy(t_hbm.at[idx_v.at[c]], rows, sem).wait()
        pltpu.sync_copy(rows, out_hbm.at[pl.ds(wid * ROWS_PER_WORKER + c * CHUNK, CHUNK)])


def _gather_add(idx_hbm, t_hbm, part_hbm, out_hbm, idx_v, rows, prev, acc, sem):
    wid = lax.axis_index("s") * 2 + lax.axis_index("c")
    pltpu.sync_copy(idx_hbm.at[wid], idx_v)
    for c in range(CHUNKS_PER_WORKER):
        base = wid * ROWS_PER_WORKER + c * CHUNK
        cp = pltpu.async_copy(t_hbm.at[idx_v.at[c]], rows, sem)
        pltpu.sync_copy(part_hbm.at[pl.ds(base, CHUNK)], prev)
        cp.wait()

        def row_body(row, _):
            for cc in range(EMB_DIM // LANES):
                s = pl.ds(cc * LANES, LANES)
                acc[row, s] = rows[row, s] + prev[row, s]
            return 0

        lax.fori_loop(0, CHUNK, row_body, 0)
        pltpu.sync_copy(acc, out_hbm.at[pl.ds(base, CHUNK)])


def kernel(indices, T0, T1, T2):
    idx_r = indices.T.reshape(NUM_TABLES, NUM_WORKERS, CHUNKS_PER_WORKER, CHUNK)

    mesh = plsc.VectorSubcoreMesh(core_axis_name="c", subcore_axis_name="s")
    params = pltpu.CompilerParams(use_tc_tiling_on_sc=False)
    out_t = jax.ShapeDtypeStruct((BATCH_SIZE, EMB_DIM), jnp.float32)
    idx_scr = pltpu.VMEM((CHUNKS_PER_WORKER, CHUNK), jnp.int32)
    row_scr = pltpu.VMEM((CHUNK, EMB_DIM), jnp.float32)

    first = functools.partial(
        pl.kernel, mesh=mesh, compiler_params=params, out_type=out_t,
        scratch_types=[idx_scr, row_scr, pltpu.SemaphoreType.DMA],
    )(_gather_first)
    step = functools.partial(
        pl.kernel, mesh=mesh, compiler_params=params, out_type=out_t,
        scratch_types=[idx_scr, row_scr, row_scr, row_scr,
                       pltpu.SemaphoreType.DMA],
    )(_gather_add)

    part = first(idx_r[0], T0)
    part = step(idx_r[1], T1, part)
    return step(idx_r[2], T2, part)


# speedup vs baseline: 1.5280x; 1.5280x over previous
import functools

import jax
import jax.numpy as jnp
from jax import lax
from jax.experimental import pallas as pl
from jax.experimental.pallas import tpu as pltpu
from jax.experimental.pallas import tpu_sc as plsc

NUM_TABLES = 3
EMB_DIM = 64
BATCH_SIZE = 16384
NUM_WORKERS = 32
ROWS_PER_WORKER = BATCH_SIZE // NUM_WORKERS
CHUNK = 128
CHUNKS_PER_WORKER = ROWS_PER_WORKER // CHUNK
LANES = 16
GROUPS = CHUNK // LANES


def _sc_body(idx_hbm, t0_hbm, t1_hbm, t2_hbm, out_hbm,
             idx_vm, r0, r1, r2, acc, sem):
    wid = lax.axis_index("s") * 2 + lax.axis_index("c")
    pltpu.sync_copy(idx_hbm.at[wid], idx_vm)
    lane_iota = lax.iota(jnp.int32, LANES)
    tables = (t0_hbm, t1_hbm, t2_hbm)
    bufs = (r0, r1, r2)
    for k in range(CHUNKS_PER_WORKER):

        def fire_group(g, _):
            vecs = [idx_vm[t, k, pl.ds(g * LANES, LANES)]
                    for t in range(NUM_TABLES)]
            for i in range(LANES):
                for t in range(NUM_TABLES):
                    row = jnp.sum(jnp.where(lane_iota == i, vecs[t], 0))
                    pltpu.async_copy(tables[t].at[pl.ds(row, 1)],
                                     bufs[t].at[pl.ds(g * LANES + i, 1)], sem)
            return 0

        lax.fori_loop(0, GROUPS, fire_group, 0)
        pltpu.make_async_copy(t0_hbm.at[pl.ds(0, CHUNK)], r0, sem).wait()
        pltpu.make_async_copy(t1_hbm.at[pl.ds(0, CHUNK)], r1, sem).wait()
        pltpu.make_async_copy(t2_hbm.at[pl.ds(0, CHUNK)], r2, sem).wait()

        def row_body(row, _):
            for cc in range(EMB_DIM // LANES):
                s = pl.ds(cc * LANES, LANES)
                acc[row, s] = r0[row, s] + r1[row, s] + r2[row, s]
            return 0

        lax.fori_loop(0, CHUNK, row_body, 0)
        pltpu.sync_copy(acc, out_hbm.at[pl.ds(wid * ROWS_PER_WORKER + k * CHUNK, CHUNK)])


def kernel(indices, T0, T1, T2):
    idx_r = indices.T.reshape(NUM_TABLES, NUM_WORKERS, CHUNKS_PER_WORKER, CHUNK)
    idx_r = idx_r.transpose(1, 0, 2, 3)

    mesh = plsc.VectorSubcoreMesh(core_axis_name="c", subcore_axis_name="s")
    run = functools.partial(
        pl.kernel,
        mesh=mesh,
        compiler_params=pltpu.CompilerParams(needs_layout_passes=False),
        out_type=jax.ShapeDtypeStruct((BATCH_SIZE, EMB_DIM), jnp.float32),
        scratch_types=[
            pltpu.VMEM((NUM_TABLES, CHUNKS_PER_WORKER, CHUNK), jnp.int32),
            pltpu.VMEM((CHUNK, EMB_DIM), jnp.float32),
            pltpu.VMEM((CHUNK, EMB_DIM), jnp.float32),
            pltpu.VMEM((CHUNK, EMB_DIM), jnp.float32),
            pltpu.VMEM((CHUNK, EMB_DIM), jnp.float32),
            pltpu.SemaphoreType.DMA,
        ],
    )(_sc_body)
    return run(idx_r, T0, T1, T2)
